# baseline (speedup 1.0000x reference)
import functools

import jax
import jax.numpy as jnp
from jax import lax
from jax.experimental import pallas as pl
from jax.experimental.pallas import tpu as pltpu
from jax.experimental.pallas import tpu_sc as plsc

B, N, E = 2, 10000, 160000
IN_DIM, OUT_DIM, E_H, E_OUT = 128, 128, 32, 30

NC, NS = 2, 16
NW = NC * NS
WIN = 128
KPW = 80
WING = 512
KPWG = 20
EP = NS * KPW * WIN
PAD = EP - E
BEP = B * EP
NACC = 10112
RPS = NACC // NS

NCHUNK = 2
EPC = EP // NCHUNK
GCC = B * EPC // NW
KPWGC = GCC // WING
KPWC = EPC // (NS * WIN)
UPW = 20


def _sig(z):
    return 1.0 / (1.0 + jnp.exp(-z))


def _pq_body(x_ref, w1s_ref, w1t_ref, b1_ref, p_ref, q_ref):
    xb = x_ref[...]
    p_ref[...] = jnp.dot(xb, w1s_ref[...],
                         preferred_element_type=jnp.float32) + b1_ref[...]
    q_ref[...] = jnp.dot(xb, w1t_ref[...], preferred_element_type=jnp.float32)


def _pq_tables(x4, w1s4, w1t4, b1r4):
    blk = 1000
    rows = (B * N) // 4
    grid = rows // blk
    return pl.pallas_call(
        _pq_body,
        grid=(grid,),
        in_specs=[
            pl.BlockSpec((blk, 4 * IN_DIM), lambda i: (i, 0)),
            pl.BlockSpec((4 * IN_DIM, 128), lambda i: (0, 0)),
            pl.BlockSpec((4 * IN_DIM, 128), lambda i: (0, 0)),
            pl.BlockSpec((1, 128), lambda i: (0, 0)),
        ],
        out_specs=[
            pl.BlockSpec((blk, 128), lambda i: (i, 0)),
            pl.BlockSpec((blk, 128), lambda i: (i, 0)),
        ],
        out_shape=[
            jax.ShapeDtypeStruct((rows, 128), jnp.float32),
            jax.ShapeDtypeStruct((rows, 128), jnp.float32),
        ],
    )(x4, w1s4, w1t4, b1r4)


def _warmup_body(xt_hbm, out_hbm, t, widx, wr0, wr1, wsem0, wsem1):
    c = lax.axis_index("c")
    s = lax.axis_index("s")
    w = c * NS + s
    z16 = jnp.zeros((16,), jnp.float32)
    rw = RPS // 8

    def irow(i, _):
        v = (lax.iota(jnp.int32, 16) + i * 16 + w * 128) * 157 + w * 2503
        widx[pl.ds(i * 16, 16)] = lax.rem(v, 4 * B * N)
        return 0

    lax.fori_loop(0, 8, irow, 0)

    def spin(i, _):
        d0 = pltpu.async_copy(xt_hbm.at[widx], wr0, wsem0)
        d1 = pltpu.async_copy(xt_hbm.at[widx], wr1, wsem1)
        d0.wait()
        d1.wait()
        return 0

    lax.fori_loop(0, 12, spin, 0)

    @pl.when(s < 4)
    def _():
        v = c * 4 + s

        def zrow(i, _):
            t[i // 2, pl.ds((i % 2) * 16, 16)] = z16
            return 0

        lax.fori_loop(0, rw * 2, zrow, 0)
        pltpu.sync_copy(t, out_hbm.at[pl.ds(v * rw, rw)])


def _warmup(xt):
    mesh = plsc.VectorSubcoreMesh(core_axis_name="c", subcore_axis_name="s")
    f = pl.kernel(
        _warmup_body,
        out_type=jax.ShapeDtypeStruct((RPS, E_H), jnp.float32),
        mesh=mesh,
        compiler_params=pltpu.CompilerParams(use_tc_tiling_on_sc=False),
        scratch_types=[
            pltpu.VMEM((RPS // 8, E_H), jnp.float32),
            pltpu.VMEM((128,), jnp.int32),
            pltpu.VMEM((128, E_H), jnp.float32),
            pltpu.VMEM((128, E_H), jnp.float32),
            pltpu.SemaphoreType.DMA,
            pltpu.SemaphoreType.DMA,
        ],
    )
    return f(xt)


def _add_rows(pr, qr, nvec):
    def body(i, _):
        r = 2 * i
        pr[r, pl.ds(0, 16)] = pr[r, pl.ds(0, 16)] + qr[r, pl.ds(0, 16)]
        pr[r, pl.ds(16, 16)] = pr[r, pl.ds(16, 16)] + qr[r, pl.ds(16, 16)]
        r = 2 * i + 1
        pr[r, pl.ds(0, 16)] = pr[r, pl.ds(0, 16)] + qr[r, pl.ds(0, 16)]
        pr[r, pl.ds(16, 16)] = pr[r, pl.ds(16, 16)] + qr[r, pl.ds(16, 16)]
        return 0

    lax.fori_loop(0, nvec // 4, body, 0)


def _gather_body(p_hbm, q_hbm, gsrc_hbm, gtgt_hbm, g_hbm,
                 idxs, idxt, pr0, pr1, qr0, qr1,
                 semp0, semp1, semq0, semq1, semo0, semo1):
    c = lax.axis_index("c")
    s = lax.axis_index("s")
    w = c * NS + s
    base = w * GCC
    pltpu.sync_copy(gsrc_hbm.at[w], idxs)
    pltpu.sync_copy(gtgt_hbm.at[w], idxt)

    pr = (pr0, pr1)
    qr = (qr0, qr1)
    semp = (semp0, semp1)
    semq = (semq0, semq1)
    semo = (semo0, semo1)
    ng = KPWGC
    dp = [None, None]
    dq = [None, None]
    dout = [None, None]
    for gi in range(ng):
        b = gi % 2
        if dout[b] is not None:
            dout[b].wait()
        dp[b] = pltpu.async_copy(p_hbm.at[idxs.at[gi]], pr[b], semp[b])
        dq[b] = pltpu.async_copy(q_hbm.at[idxt.at[gi]], qr[b], semq[b])
        if gi > 0:
            pb = 1 - b
            dp[pb].wait()
            dq[pb].wait()
            _add_rows(pr[pb], qr[pb], WING * 2)
            dout[pb] = pltpu.async_copy(
                pr[pb], g_hbm.at[pl.ds(base + (gi - 1) * WING, WING)],
                semo[pb])
    lb = (ng - 1) % 2
    dp[lb].wait()
    dq[lb].wait()
    _add_rows(pr[lb], qr[lb], WING * 2)
    pltpu.sync_copy(pr[lb], g_hbm.at[pl.ds(base + (ng - 1) * WING, WING)])
    if dout[1 - lb] is not None:
        dout[1 - lb].wait()


def _gather(p, q, gsrc3, gtgt3):
    mesh = plsc.VectorSubcoreMesh(core_axis_name="c", subcore_axis_name="s")
    f = pl.kernel(
        _gather_body,
        out_type=jax.ShapeDtypeStruct((B * EPC, E_H), jnp.float32),
        mesh=mesh,
        compiler_params=pltpu.CompilerParams(use_tc_tiling_on_sc=False),
        scratch_types=[
            pltpu.VMEM((KPWGC, WING), jnp.int32),
            pltpu.VMEM((KPWGC, WING), jnp.int32),
            pltpu.VMEM((WING, E_H), jnp.float32),
            pltpu.VMEM((WING, E_H), jnp.float32),
            pltpu.VMEM((WING, E_H), jnp.float32),
            pltpu.VMEM((WING, E_H), jnp.float32),
            pltpu.SemaphoreType.DMA,
            pltpu.SemaphoreType.DMA,
            pltpu.SemaphoreType.DMA,
            pltpu.SemaphoreType.DMA,
            pltpu.SemaphoreType.DMA,
            pltpu.SemaphoreType.DMA,
        ],
    )
    return f(p, q, gsrc3, gtgt3)


def _mlp_body(g_ref, w_ref, a_ref, bw_ref, w2_ref, b2_ref, u_ref):
    wv = w_ref[...]
    g1 = (wv[:, None, :] * a_ref[...][None, :, :]).reshape(1024, 128)
    z = g_ref[...] + jnp.dot(g1, bw_ref[...],
                             preferred_element_type=jnp.float32)
    h1 = _sig(z)
    u_ref[...] = _sig(jnp.dot(h1, w2_ref[...],
                              preferred_element_type=jnp.float32) + b2_ref[...])


def _edge_mlp(g4, wpk, amat, bwmat, w2bd, b2bd):
    blk = 1024
    rows = B * EPC // 4
    grid = rows // blk
    return pl.pallas_call(
        _mlp_body,
        grid=(grid,),
        in_specs=[
            pl.BlockSpec((blk, 128), lambda i: (i, 0)),
            pl.BlockSpec((blk // 32, 128), lambda i: (i, 0)),
            pl.BlockSpec((32, 128), lambda i: (0, 0)),
            pl.BlockSpec((128, 128), lambda i: (0, 0)),
            pl.BlockSpec((128, 128), lambda i: (0, 0)),
            pl.BlockSpec((1, 128), lambda i: (0, 0)),
        ],
        out_specs=pl.BlockSpec((blk, 128), lambda i: (i, 0)),
        out_shape=jax.ShapeDtypeStruct((rows, 128), jnp.float32),
    )(g4, wpk, amat, bwmat, w2bd, b2bd)


def _scatter_body(u_hbm, sidx0_hbm, sidx1_hbm, zeros_hbm, out_hbm,
                  idx0, idx1, rows0, rows1, t0, t1, acc, acc2,
                  si0, si1, ss00, ss01, ss10, ss11):
    c = lax.axis_index("c")
    s = lax.axis_index("s")
    sl = pl.ds(s * RPS, RPS)
    pltpu.sync_copy(zeros_hbm, acc.at[sl])
    pltpu.sync_copy(zeros_hbm, acc2.at[sl])
    plsc.subcore_barrier()
    base = c * EPC + s * (KPWC * WIN)
    pltpu.sync_copy(sidx0_hbm.at[c * NS + s], idx0)
    pltpu.sync_copy(sidx1_hbm.at[c * NS + s], idx1)

    rows = (rows0, rows1)
    semi = (si0, si1)
    sems0 = (ss00, ss01)
    sems1 = (ss10, ss11)
    upw = UPW

    def supergroup(sg, _):
        k0 = sg * upw
        din = [None, None]
        ds_ = [None, None]
        din[0] = pltpu.async_copy(u_hbm.at[pl.ds(base + k0 * WIN, WIN)],
                                  rows[0], semi[0])
        for j in range(upw):
            b = j % 2
            nb = 1 - b
            din[b].wait()
            if j + 1 < upw:
                if ds_[nb] is not None:
                    ds_[nb][0].wait()
                    ds_[nb][1].wait()
                din[nb] = pltpu.async_copy(
                    u_hbm.at[pl.ds(base + (k0 + j + 1) * WIN, WIN)],
                    rows[nb], semi[nb])
            ds_[b] = (
                pltpu.async_copy(rows[b], acc.at[idx0.at[k0 + j]],
                                 sems0[b], add=True),
                pltpu.async_copy(rows[b], acc2.at[idx1.at[k0 + j]],
                                 sems1[b], add=True),
            )
        for b in range(2):
            if ds_[b] is not None:
                ds_[b][0].wait()
                ds_[b][1].wait()
        return 0

    lax.fori_loop(0, KPWC // upw, supergroup, 0)
    plsc.subcore_barrier()
    pltpu.sync_copy(acc.at[sl], t0)
    pltpu.sync_copy(acc2.at[sl], t1)

    def sub(i, _):
        r = i // 2
        h = (i % 2) * 16
        t0[r, pl.ds(h, 16)] = t0[r, pl.ds(h, 16)] - t1[r, pl.ds(h, 16)]
        return 0

    lax.fori_loop(0, RPS * 2, sub, 0)
    pltpu.sync_copy(t0, out_hbm.at[c, sl])


def _scatter(u, sidx0, sidx1, zeros):
    mesh = plsc.VectorSubcoreMesh(core_axis_name="c", subcore_axis_name="s")
    f = pl.kernel(
        _scatter_body,
        out_type=jax.ShapeDtypeStruct((B, NACC, E_H), jnp.float32),
        mesh=mesh,
        compiler_params=pltpu.CompilerParams(use_tc_tiling_on_sc=False),
        scratch_types=[
            pltpu.VMEM((KPWC, WIN), jnp.int32),
            pltpu.VMEM((KPWC, WIN), jnp.int32),
            pltpu.VMEM((WIN, E_H), jnp.float32),
            pltpu.VMEM((WIN, E_H), jnp.float32),
            pltpu.VMEM((RPS, E_H), jnp.float32),
            pltpu.VMEM((RPS, E_H), jnp.float32),
            pltpu.VMEM_SHARED((NACC, E_H), jnp.float32),
            pltpu.VMEM_SHARED((NACC, E_H), jnp.float32),
            pltpu.SemaphoreType.DMA,
            pltpu.SemaphoreType.DMA,
            pltpu.SemaphoreType.DMA,
            pltpu.SemaphoreType.DMA,
            pltpu.SemaphoreType.DMA,
            pltpu.SemaphoreType.DMA,
        ],
    )
    return f(u, sidx0, sidx1, zeros)


def _out_body(a_ref, a2_ref, w3_ref, b3_ref, y_ref):
    a = a_ref[0] + a2_ref[0]
    y_ref[0] = _sig(jnp.dot(a, w3_ref[...],
                            preferred_element_type=jnp.float32) + b3_ref[...])


def _final(accs, w3p, b3r):
    blk = 1000
    grid = N // blk
    aspec = pl.BlockSpec((1, blk, E_H), lambda b, i: (b, i, 0))
    return pl.pallas_call(
        _out_body,
        grid=(B, grid),
        in_specs=[
            aspec, aspec,
            pl.BlockSpec((E_H, OUT_DIM), lambda b, i: (0, 0)),
            pl.BlockSpec((1, OUT_DIM), lambda b, i: (0, 0)),
        ],
        out_specs=pl.BlockSpec((1, blk, OUT_DIM), lambda b, i: (b, i, 0)),
        out_shape=jax.ShapeDtypeStruct((B, N, OUT_DIM), jnp.float32),
    )(*accs, w3p, b3r)


def kernel(x, edge_src_target, edge_weight, W1, b1, W2, b2, W3, b3):
    src = edge_src_target[0]
    tgt = edge_src_target[1]

    eye4 = jnp.eye(4, dtype=jnp.float32)
    w1s4 = jnp.kron(eye4, W1[:IN_DIM])
    w1t4 = jnp.kron(eye4, W1[IN_DIM:2 * IN_DIM])
    w1w = W1[2 * IN_DIM]
    b1r4 = jnp.tile(b1, 4).reshape(1, 128)
    k128 = jnp.arange(128)
    amat = (k128[None, :] // 4 == jnp.arange(32)[:, None]).astype(jnp.float32)
    bwmat = ((k128[:, None] % 4) == (k128[None, :] // E_H)).astype(
        jnp.float32) * jnp.tile(w1w, 4)[None, :]
    w2p = jnp.pad(W2, ((0, 0), (0, E_H - E_OUT)))
    w2bd = jnp.kron(eye4, w2p)
    b2bd = jnp.tile(jnp.pad(b2, (0, E_H - E_OUT)), 4).reshape(1, 128)
    w3p = jnp.pad(W3, ((0, E_H - E_OUT), (0, 0)))
    b3r = b3.reshape(1, OUT_DIM)

    ipad = jnp.zeros((PAD,), jnp.int32)
    src_p = jnp.concatenate([src, ipad])
    tgt_p = jnp.concatenate([tgt, ipad])
    boffs = (jnp.arange(B, dtype=jnp.int32) * N)[:, None]
    dump = N + (jnp.arange(PAD, dtype=jnp.int32) % NS)
    stgt = jnp.concatenate([tgt, dump])
    ssrc = jnp.concatenate([src, dump])
    wpad = jnp.concatenate([edge_weight, jnp.zeros((PAD,), jnp.float32)])

    zeros = _warmup(x.reshape(4 * B * N, E_H))
    zi = zeros[0, 0].astype(jnp.int32)

    x4 = x.reshape(B * N // 4, 4 * IN_DIM)
    p4, q4 = _pq_tables(x4, w1s4, w1t4, b1r4)
    p = p4.reshape(B * N, E_H)
    q = q4.reshape(B * N, E_H)

    accs = []
    for k in range(NCHUNK):
        ck = slice(k * EPC, (k + 1) * EPC)
        boffs_k = boffs + zi if k == 0 else boffs
        gsrc3 = (src_p[ck][None, :] + boffs_k).reshape(NW, KPWGC, WING)
        gtgt3 = (tgt_p[ck][None, :] + boffs).reshape(NW, KPWGC, WING)
        sidx0 = jnp.broadcast_to(stgt[ck][None, :],
                                 (B, EPC)).reshape(B * NS, KPWC, WIN)
        sidx1 = jnp.broadcast_to(ssrc[ck][None, :],
                                 (B, EPC)).reshape(B * NS, KPWC, WIN)
        wpk = jnp.broadcast_to(wpad[ck][None, :],
                               (B, EPC)).reshape(B * EPC // 128, 128)
        g = _gather(p, q, gsrc3, gtgt3)
        u4 = _edge_mlp(g.reshape(B * EPC // 4, 128),
                       wpk, amat, bwmat, w2bd, b2bd)
        accs.append(_scatter(u4.reshape(B * EPC, E_H), sidx0, sidx1, zeros))
    return _final(accs, w3p, b3r)


# speedup vs baseline: 1.0558x; 1.0558x over previous
import functools

import jax
import jax.numpy as jnp
from jax import lax
from jax.experimental import pallas as pl
from jax.experimental.pallas import tpu as pltpu
from jax.experimental.pallas import tpu_sc as plsc

B, N, E = 2, 10000, 160000
IN_DIM, OUT_DIM, E_H, E_OUT = 128, 128, 32, 30

NC, NS = 2, 16
NW = NC * NS
WIN = 128
KPW = 80
WING = 512
KPWG = 20
EP = NS * KPW * WIN
PAD = EP - E
BEP = B * EP
NACC = 10112
RPS = NACC // NS

NCHUNK = 2
EPC = EP // NCHUNK
GCC = B * EPC // NW
KPWGC = GCC // WING
KPWC = EPC // (NS * WIN)
UPW = 20


def _sig(z):
    return 1.0 / (1.0 + jnp.exp(-z))


def _pq_body(x_ref, w1s_ref, w1t_ref, b1_ref, p_ref, q_ref):
    xb = x_ref[...]
    p_ref[...] = jnp.dot(xb, w1s_ref[...],
                         preferred_element_type=jnp.float32) + b1_ref[...]
    q_ref[...] = jnp.dot(xb, w1t_ref[...], preferred_element_type=jnp.float32)


def _pq_tables(x4, w1s4, w1t4, b1r4):
    blk = 1000
    rows = (B * N) // 4
    grid = rows // blk
    return pl.pallas_call(
        _pq_body,
        grid=(grid,),
        in_specs=[
            pl.BlockSpec((blk, 4 * IN_DIM), lambda i: (i, 0)),
            pl.BlockSpec((4 * IN_DIM, 128), lambda i: (0, 0)),
            pl.BlockSpec((4 * IN_DIM, 128), lambda i: (0, 0)),
            pl.BlockSpec((1, 128), lambda i: (0, 0)),
        ],
        out_specs=[
            pl.BlockSpec((blk, 128), lambda i: (i, 0)),
            pl.BlockSpec((blk, 128), lambda i: (i, 0)),
        ],
        out_shape=[
            jax.ShapeDtypeStruct((rows, 128), jnp.float32),
            jax.ShapeDtypeStruct((rows, 128), jnp.float32),
        ],
    )(x4, w1s4, w1t4, b1r4)


def _warmup_body(dep_hbm, out_hbm, t, widx, wrows, wsem):
    c = lax.axis_index("c")
    s = lax.axis_index("s")
    w = c * NS + s
    z16 = jnp.zeros((16,), jnp.float32)
    rw = RPS // 8

    def irow(i, _):
        widx[pl.ds(i * 16, 16)] = lax.iota(jnp.int32, 16) + i * 16 + w * 128
        return 0

    lax.fori_loop(0, 8, irow, 0)
    pltpu.async_copy(dep_hbm.at[widx], wrows, wsem).wait()

    @pl.when(s < 4)
    def _():
        v = c * 4 + s

        def zrow(i, _):
            t[i // 2, pl.ds((i % 2) * 16, 16)] = z16
            return 0

        lax.fori_loop(0, rw * 2, zrow, 0)
        pltpu.sync_copy(t, out_hbm.at[pl.ds(v * rw, rw)])


def _warmup(dep):
    mesh = plsc.VectorSubcoreMesh(core_axis_name="c", subcore_axis_name="s")
    f = pl.kernel(
        _warmup_body,
        out_type=jax.ShapeDtypeStruct((RPS, E_H), jnp.float32),
        mesh=mesh,
        compiler_params=pltpu.CompilerParams(use_tc_tiling_on_sc=False),
        scratch_types=[
            pltpu.VMEM((RPS // 8, E_H), jnp.float32),
            pltpu.VMEM((128,), jnp.int32),
            pltpu.VMEM((128, E_H), jnp.float32),
            pltpu.SemaphoreType.DMA,
        ],
    )
    return f(dep)


def _add_rows(pr, qr, nvec):
    def body(i, _):
        r = 2 * i
        pr[r, pl.ds(0, 16)] = pr[r, pl.ds(0, 16)] + qr[r, pl.ds(0, 16)]
        pr[r, pl.ds(16, 16)] = pr[r, pl.ds(16, 16)] + qr[r, pl.ds(16, 16)]
        r = 2 * i + 1
        pr[r, pl.ds(0, 16)] = pr[r, pl.ds(0, 16)] + qr[r, pl.ds(0, 16)]
        pr[r, pl.ds(16, 16)] = pr[r, pl.ds(16, 16)] + qr[r, pl.ds(16, 16)]
        return 0

    lax.fori_loop(0, nvec // 4, body, 0)


def _gather_body(p_hbm, q_hbm, gsrc_hbm, gtgt_hbm, g_hbm,
                 idxs, idxt, pr0, pr1, qr0, qr1,
                 semp0, semp1, semq0, semq1, semo0, semo1):
    c = lax.axis_index("c")
    s = lax.axis_index("s")
    w = c * NS + s
    base = w * GCC
    pltpu.sync_copy(gsrc_hbm.at[w], idxs)
    pltpu.sync_copy(gtgt_hbm.at[w], idxt)

    pr = (pr0, pr1)
    qr = (qr0, qr1)
    semp = (semp0, semp1)
    semq = (semq0, semq1)
    semo = (semo0, semo1)
    ng = KPWGC
    dp = [None, None]
    dq = [None, None]
    dout = [None, None]
    for gi in range(ng):
        b = gi % 2
        if dout[b] is not None:
            dout[b].wait()
        dp[b] = pltpu.async_copy(p_hbm.at[idxs.at[gi]], pr[b], semp[b])
        dq[b] = pltpu.async_copy(q_hbm.at[idxt.at[gi]], qr[b], semq[b])
        if gi > 0:
            pb = 1 - b
            dp[pb].wait()
            dq[pb].wait()
            _add_rows(pr[pb], qr[pb], WING * 2)
            dout[pb] = pltpu.async_copy(
                pr[pb], g_hbm.at[pl.ds(base + (gi - 1) * WING, WING)],
                semo[pb])
    lb = (ng - 1) % 2
    dp[lb].wait()
    dq[lb].wait()
    _add_rows(pr[lb], qr[lb], WING * 2)
    pltpu.sync_copy(pr[lb], g_hbm.at[pl.ds(base + (ng - 1) * WING, WING)])
    if dout[1 - lb] is not None:
        dout[1 - lb].wait()


def _gather(p, q, gsrc3, gtgt3):
    mesh = plsc.VectorSubcoreMesh(core_axis_name="c", subcore_axis_name="s")
    f = pl.kernel(
        _gather_body,
        out_type=jax.ShapeDtypeStruct((B * EPC, E_H), jnp.float32),
        mesh=mesh,
        compiler_params=pltpu.CompilerParams(use_tc_tiling_on_sc=False),
        scratch_types=[
            pltpu.VMEM((KPWGC, WING), jnp.int32),
            pltpu.VMEM((KPWGC, WING), jnp.int32),
            pltpu.VMEM((WING, E_H), jnp.float32),
            pltpu.VMEM((WING, E_H), jnp.float32),
            pltpu.VMEM((WING, E_H), jnp.float32),
            pltpu.VMEM((WING, E_H), jnp.float32),
            pltpu.SemaphoreType.DMA,
            pltpu.SemaphoreType.DMA,
            pltpu.SemaphoreType.DMA,
            pltpu.SemaphoreType.DMA,
            pltpu.SemaphoreType.DMA,
            pltpu.SemaphoreType.DMA,
        ],
    )
    return f(p, q, gsrc3, gtgt3)


def _mlp_body(g_ref, w_ref, a_ref, bw_ref, w2_ref, b2_ref, u_ref):
    wv = w_ref[...]
    g1 = (wv[:, None, :] * a_ref[...][None, :, :]).reshape(1024, 128)
    z = g_ref[...] + jnp.dot(g1, bw_ref[...],
                             preferred_element_type=jnp.float32)
    h1 = _sig(z)
    u_ref[...] = _sig(jnp.dot(h1, w2_ref[...],
                              preferred_element_type=jnp.float32) + b2_ref[...])


def _edge_mlp(g4, wpk, amat, bwmat, w2bd, b2bd):
    blk = 1024
    rows = B * EPC // 4
    grid = rows // blk
    return pl.pallas_call(
        _mlp_body,
        grid=(grid,),
        in_specs=[
            pl.BlockSpec((blk, 128), lambda i: (i, 0)),
            pl.BlockSpec((blk // 32, 128), lambda i: (i, 0)),
            pl.BlockSpec((32, 128), lambda i: (0, 0)),
            pl.BlockSpec((128, 128), lambda i: (0, 0)),
            pl.BlockSpec((128, 128), lambda i: (0, 0)),
            pl.BlockSpec((1, 128), lambda i: (0, 0)),
        ],
        out_specs=pl.BlockSpec((blk, 128), lambda i: (i, 0)),
        out_shape=jax.ShapeDtypeStruct((rows, 128), jnp.float32),
    )(g4, wpk, amat, bwmat, w2bd, b2bd)


def _scatter_body(u_hbm, sidx0_hbm, sidx1_hbm, zeros_hbm, out_hbm,
                  idx0, idx1, rows0, rows1, t0, t1, acc, acc2,
                  si0, si1, ss00, ss01, ss10, ss11):
    c = lax.axis_index("c")
    s = lax.axis_index("s")
    sl = pl.ds(s * RPS, RPS)
    pltpu.sync_copy(zeros_hbm, acc.at[sl])
    pltpu.sync_copy(zeros_hbm, acc2.at[sl])
    plsc.subcore_barrier()
    base = c * EPC + s * (KPWC * WIN)
    pltpu.sync_copy(sidx0_hbm.at[c * NS + s], idx0)
    pltpu.sync_copy(sidx1_hbm.at[c * NS + s], idx1)

    rows = (rows0, rows1)
    semi = (si0, si1)
    sems0 = (ss00, ss01)
    sems1 = (ss10, ss11)
    upw = UPW

    def supergroup(sg, _):
        k0 = sg * upw
        din = [None, None]
        ds_ = [None, None]
        din[0] = pltpu.async_copy(u_hbm.at[pl.ds(base + k0 * WIN, WIN)],
                                  rows[0], semi[0])
        for j in range(upw):
            b = j % 2
            nb = 1 - b
            din[b].wait()
            if j + 1 < upw:
                if ds_[nb] is not None:
                    ds_[nb][0].wait()
                    ds_[nb][1].wait()
                din[nb] = pltpu.async_copy(
                    u_hbm.at[pl.ds(base + (k0 + j + 1) * WIN, WIN)],
                    rows[nb], semi[nb])
            ds_[b] = (
                pltpu.async_copy(rows[b], acc.at[idx0.at[k0 + j]],
                                 sems0[b], add=True),
                pltpu.async_copy(rows[b], acc2.at[idx1.at[k0 + j]],
                                 sems1[b], add=True),
            )
        for b in range(2):
            if ds_[b] is not None:
                ds_[b][0].wait()
                ds_[b][1].wait()
        return 0

    lax.fori_loop(0, KPWC // upw, supergroup, 0)
    plsc.subcore_barrier()
    pltpu.sync_copy(acc.at[sl], t0)
    pltpu.sync_copy(acc2.at[sl], t1)

    def sub(i, _):
        r = i // 2
        h = (i % 2) * 16
        t0[r, pl.ds(h, 16)] = t0[r, pl.ds(h, 16)] - t1[r, pl.ds(h, 16)]
        return 0

    lax.fori_loop(0, RPS * 2, sub, 0)
    pltpu.sync_copy(t0, out_hbm.at[c, sl])


def _scatter(u, sidx0, sidx1, zeros):
    mesh = plsc.VectorSubcoreMesh(core_axis_name="c", subcore_axis_name="s")
    f = pl.kernel(
        _scatter_body,
        out_type=jax.ShapeDtypeStruct((B, NACC, E_H), jnp.float32),
        mesh=mesh,
        compiler_params=pltpu.CompilerParams(use_tc_tiling_on_sc=False),
        scratch_types=[
            pltpu.VMEM((KPWC, WIN), jnp.int32),
            pltpu.VMEM((KPWC, WIN), jnp.int32),
            pltpu.VMEM((WIN, E_H), jnp.float32),
            pltpu.VMEM((WIN, E_H), jnp.float32),
            pltpu.VMEM((RPS, E_H), jnp.float32),
            pltpu.VMEM((RPS, E_H), jnp.float32),
            pltpu.VMEM_SHARED((NACC, E_H), jnp.float32),
            pltpu.VMEM_SHARED((NACC, E_H), jnp.float32),
            pltpu.SemaphoreType.DMA,
            pltpu.SemaphoreType.DMA,
            pltpu.SemaphoreType.DMA,
            pltpu.SemaphoreType.DMA,
            pltpu.SemaphoreType.DMA,
            pltpu.SemaphoreType.DMA,
        ],
    )
    return f(u, sidx0, sidx1, zeros)


def _out_body(a_ref, a2_ref, w3_ref, b3_ref, y_ref):
    a = a_ref[0] + a2_ref[0]
    y_ref[0] = _sig(jnp.dot(a, w3_ref[...],
                            preferred_element_type=jnp.float32) + b3_ref[...])


def _final(accs, w3p, b3r):
    blk = 1000
    grid = N // blk
    aspec = pl.BlockSpec((1, blk, E_H), lambda b, i: (b, i, 0))
    return pl.pallas_call(
        _out_body,
        grid=(B, grid),
        in_specs=[
            aspec, aspec,
            pl.BlockSpec((E_H, OUT_DIM), lambda b, i: (0, 0)),
            pl.BlockSpec((1, OUT_DIM), lambda b, i: (0, 0)),
        ],
        out_specs=pl.BlockSpec((1, blk, OUT_DIM), lambda b, i: (b, i, 0)),
        out_shape=jax.ShapeDtypeStruct((B, N, OUT_DIM), jnp.float32),
    )(*accs, w3p, b3r)


def kernel(x, edge_src_target, edge_weight, W1, b1, W2, b2, W3, b3):
    src = edge_src_target[0]
    tgt = edge_src_target[1]

    eye4 = jnp.eye(4, dtype=jnp.float32)
    w1s4 = jnp.kron(eye4, W1[:IN_DIM])
    w1t4 = jnp.kron(eye4, W1[IN_DIM:2 * IN_DIM])
    w1w = W1[2 * IN_DIM]
    b1r4 = jnp.tile(b1, 4).reshape(1, 128)
    k128 = jnp.arange(128)
    amat = (k128[None, :] // 4 == jnp.arange(32)[:, None]).astype(jnp.float32)
    bwmat = ((k128[:, None] % 4) == (k128[None, :] // E_H)).astype(
        jnp.float32) * jnp.tile(w1w, 4)[None, :]
    w2p = jnp.pad(W2, ((0, 0), (0, E_H - E_OUT)))
    w2bd = jnp.kron(eye4, w2p)
    b2bd = jnp.tile(jnp.pad(b2, (0, E_H - E_OUT)), 4).reshape(1, 128)
    w3p = jnp.pad(W3, ((0, E_H - E_OUT), (0, 0)))
    b3r = b3.reshape(1, OUT_DIM)

    ipad = jnp.zeros((PAD,), jnp.int32)
    src_p = jnp.concatenate([src, ipad])
    tgt_p = jnp.concatenate([tgt, ipad])
    boffs = (jnp.arange(B, dtype=jnp.int32) * N)[:, None]
    dump = N + (jnp.arange(PAD, dtype=jnp.int32) % NS)
    stgt = jnp.concatenate([tgt, dump])
    ssrc = jnp.concatenate([src, dump])
    wpad = jnp.concatenate([edge_weight, jnp.zeros((PAD,), jnp.float32)])

    x4 = x.reshape(B * N // 4, 4 * IN_DIM)
    p4, q4 = _pq_tables(x4, w1s4, w1t4, b1r4)
    p = p4.reshape(B * N, E_H)
    q = q4.reshape(B * N, E_H)
    zeros = _warmup(p)
    zi = zeros[0, 0].astype(jnp.int32)

    accs = []
    for k in range(NCHUNK):
        ck = slice(k * EPC, (k + 1) * EPC)
        boffs_k = boffs + zi if k == 0 else boffs
        gsrc3 = (src_p[ck][None, :] + boffs_k).reshape(NW, KPWGC, WING)
        gtgt3 = (tgt_p[ck][None, :] + boffs).reshape(NW, KPWGC, WING)
        sidx0 = jnp.broadcast_to(stgt[ck][None, :],
                                 (B, EPC)).reshape(B * NS, KPWC, WIN)
        sidx1 = jnp.broadcast_to(ssrc[ck][None, :],
                                 (B, EPC)).reshape(B * NS, KPWC, WIN)
        wpk = jnp.broadcast_to(wpad[ck][None, :],
                               (B, EPC)).reshape(B * EPC // 128, 128)
        g = _gather(p, q, gsrc3, gtgt3)
        u4 = _edge_mlp(g.reshape(B * EPC // 4, 128),
                       wpk, amat, bwmat, w2bd, b2bd)
        accs.append(_scatter(u4.reshape(B * EPC, E_H), sidx0, sidx1, zeros))
    return _final(accs, w3p, b3r)
